# baseline (speedup 1.0000x reference)
import jax
import jax.numpy as jnp
from jax.experimental import pallas as pl
from jax.experimental.pallas import tpu as pltpu

B, T, S, C, V, D = 16, 128, 512, 512, 50000, 1024
BT = B * T
NEG = -1e30

BN = 1024
VP = 51200
KB = VP // BN

CB = 49152
CW = 2048
COFF = V - CB

RB = 64


def _gate_copy_kernel(hid_ref, attn_ref, sm_ref, wc_ref, bc_ref,
                      pc_ref, cp_ref):
    pc = jax.nn.sigmoid(
        jnp.dot(hid_ref[...], wc_ref[...], preferred_element_type=jnp.float32)
        + bc_ref[0, 0])
    pc_ref[...] = jnp.broadcast_to(pc, (T, 128))
    mula = attn_ref[...] * pc
    cp = jnp.dot(mula, sm_ref[0], preferred_element_type=jnp.float32)
    cp_ref[...] = jnp.concatenate(
        [jnp.zeros((T, COFF), jnp.float32), cp,
         jnp.zeros((T, CW - COFF - C), jnp.float32)], axis=1)


def _logits_kernel(hid_ref, w_ref, b_ref, lg_ref, m_ref, s_ref, m_s, s_s):
    k = pl.program_id(0)

    @pl.when(k == 0)
    def _():
        m_s[...] = jnp.full((BT, 1), NEG, jnp.float32)
        s_s[...] = jnp.zeros((BT, 1), jnp.float32)

    l = jax.lax.dot_general(hid_ref[...], w_ref[...],
                            (((1,), (1,)), ((), ())),
                            preferred_element_type=jnp.float32)
    l = l + b_ref[...]
    col = jax.lax.broadcasted_iota(jnp.int32, (1, BN), 1) + k * BN
    l = jnp.where(col >= V, NEG, l)

    tmax = jnp.max(l, axis=1, keepdims=True)
    m_old = m_s[...]
    m_new = jnp.maximum(m_old, tmax)
    s_new = (s_s[...] * jnp.exp(m_old - m_new)
             + jnp.sum(jnp.exp(l - m_new), axis=1, keepdims=True))
    m_s[...] = m_new
    s_s[...] = s_new

    lg_ref[...] = l.astype(jnp.bfloat16)
    m_ref[...] = jnp.broadcast_to(m_new, (BT, 128))
    s_ref[...] = jnp.broadcast_to(s_new, (BT, 128))


def _finalize_kernel(lg_ref, m_ref, s_ref, pc_ref, cp_ref, out_ref):
    m0 = jnp.max(m_ref[...], axis=1, keepdims=True)
    s0 = jnp.max(s_ref[...], axis=1, keepdims=True)
    pc0 = jnp.max(pc_ref[...], axis=1, keepdims=True)
    scale = (1.0 - pc0) / s0
    val = jnp.exp(lg_ref[...].astype(jnp.float32) - m0) * scale
    out_ref[:, :CB] = val[:, :CB]
    out_ref[:, CB:] = val[:, CB:V + C] + cp_ref[...][:, :V + C - CB]


def kernel(hidden, attn, src_map, W, b, w_copy, b_copy, pad_idx):
    b_m = b.at[pad_idx].set(NEG)
    b_ext = jnp.concatenate(
        [b_m, jnp.zeros((VP - V,), jnp.float32)]).reshape(1, VP)
    wc = w_copy.reshape(D, 1)
    bc = b_copy.reshape(1, 1)

    pc, cp = pl.pallas_call(
        _gate_copy_kernel,
        grid=(B,),
        in_specs=[
            pl.BlockSpec((T, D), lambda i: (i, 0)),
            pl.BlockSpec((T, S), lambda i: (i, 0)),
            pl.BlockSpec((1, S, C), lambda i: (i, 0, 0)),
            pl.BlockSpec((D, 1), lambda i: (0, 0)),
            pl.BlockSpec((1, 1), lambda i: (0, 0)),
        ],
        out_specs=[
            pl.BlockSpec((T, 128), lambda i: (i, 0)),
            pl.BlockSpec((T, CW), lambda i: (i, 0)),
        ],
        out_shape=[
            jax.ShapeDtypeStruct((BT, 128), jnp.float32),
            jax.ShapeDtypeStruct((BT, CW), jnp.float32),
        ],
        compiler_params=pltpu.CompilerParams(
            dimension_semantics=("arbitrary",)),
    )(hidden, attn, src_map, wc, bc)

    lg, m, s = pl.pallas_call(
        _logits_kernel,
        grid=(KB,),
        in_specs=[
            pl.BlockSpec((BT, D), lambda k: (0, 0)),
            pl.BlockSpec((BN, D), lambda k: (jnp.minimum(k, V // BN), 0)),
            pl.BlockSpec((1, BN), lambda k: (0, k)),
        ],
        out_specs=[
            pl.BlockSpec((BT, BN), lambda k: (0, k)),
            pl.BlockSpec((BT, 128), lambda k: (0, 0)),
            pl.BlockSpec((BT, 128), lambda k: (0, 0)),
        ],
        out_shape=[
            jax.ShapeDtypeStruct((BT, VP), jnp.bfloat16),
            jax.ShapeDtypeStruct((BT, 128), jnp.float32),
            jax.ShapeDtypeStruct((BT, 128), jnp.float32),
        ],
        scratch_shapes=[
            pltpu.VMEM((BT, 1), jnp.float32),
            pltpu.VMEM((BT, 1), jnp.float32),
        ],
        compiler_params=pltpu.CompilerParams(
            dimension_semantics=("arbitrary",),
            vmem_limit_bytes=56 * 1024 * 1024),
    )(hidden, W, b_ext)

    out = pl.pallas_call(
        _finalize_kernel,
        grid=(BT // RB,),
        in_specs=[
            pl.BlockSpec((RB, VP), lambda i: (i, 0)),
            pl.BlockSpec((RB, 128), lambda i: (i, 0)),
            pl.BlockSpec((RB, 128), lambda i: (i, 0)),
            pl.BlockSpec((RB, 128), lambda i: (i, 0)),
            pl.BlockSpec((RB, CW), lambda i: (i, 0)),
        ],
        out_specs=pl.BlockSpec((RB, V + C), lambda i: (i, 0)),
        out_shape=jax.ShapeDtypeStruct((BT, V + C), jnp.float32),
        compiler_params=pltpu.CompilerParams(
            dimension_semantics=("arbitrary",),
            vmem_limit_bytes=56 * 1024 * 1024),
    )(lg, m, s, pc, cp)
    return out


# speedup vs baseline: 1.1118x; 1.1118x over previous
import jax
import jax.numpy as jnp
from jax.experimental import pallas as pl
from jax.experimental.pallas import tpu as pltpu

B, T, S, C, V, D = 16, 128, 512, 512, 50000, 1024
BT = B * T
NEG = -1e30

BM = 1024
BN = 2048
VP = 51200
KB = VP // BN

BNC = 2048
KC = (V + C + BNC - 1) // BNC
TB = V // BNC
OFF = V - TB * BNC


def _gate_copy_kernel(hid_ref, attn_ref, sm_ref, wc_ref, bc_ref,
                      pc_ref, cp_ref):
    pc = jax.nn.sigmoid(
        jnp.dot(hid_ref[...], wc_ref[...], preferred_element_type=jnp.float32)
        + bc_ref[0, 0])
    pc_ref[...] = jnp.broadcast_to(pc, (T, 128))
    mula = attn_ref[...] * pc
    cp = jnp.dot(mula, sm_ref[0], preferred_element_type=jnp.float32)
    cp_ref[...] = jnp.concatenate(
        [jnp.zeros((T, OFF), jnp.float32), cp,
         jnp.zeros((T, BNC - OFF - C), jnp.float32)], axis=1)


def _logits_kernel(hid_ref, w_ref, b_ref, lg_ref, m_ref, s_ref, m_s, s_s):
    k = pl.program_id(1)

    @pl.when(k == 0)
    def _():
        m_s[...] = jnp.full((BM, 1), NEG, jnp.float32)
        s_s[...] = jnp.zeros((BM, 1), jnp.float32)

    l = jax.lax.dot_general(hid_ref[...], w_ref[...],
                            (((1,), (1,)), ((), ())),
                            preferred_element_type=jnp.float32)
    l = l + b_ref[...]

    def _update(lv):
        tmax = jnp.max(lv, axis=1, keepdims=True)
        m_old = m_s[...]
        m_new = jnp.maximum(m_old, tmax)
        s_new = (s_s[...] * jnp.exp(m_old - m_new)
                 + jnp.sum(jnp.exp(lv - m_new), axis=1, keepdims=True))
        m_s[...] = m_new
        s_s[...] = s_new
        lg_ref[...] = lv.astype(jnp.bfloat16)
        return m_new, s_new

    @pl.when(k < KB - 1)
    def _():
        _update(l)

    @pl.when(k == KB - 1)
    def _():
        col = jax.lax.broadcasted_iota(jnp.int32, (1, BN), 1) + k * BN
        m_new, s_new = _update(jnp.where(col >= V, NEG, l))
        m_ref[...] = jnp.broadcast_to(m_new, (BM, 128))
        s_ref[...] = jnp.broadcast_to(s_new, (BM, 128))


def _finalize_kernel(lg_ref, m_ref, s_ref, pc_ref, cp_ref, out_ref):
    k = pl.program_id(1)
    m0 = jnp.max(m_ref[...], axis=1, keepdims=True)
    s0 = jnp.max(s_ref[...], axis=1, keepdims=True)
    pc0 = jnp.max(pc_ref[...], axis=1, keepdims=True)
    scale = (1.0 - pc0) / s0
    sm = jnp.exp(lg_ref[...].astype(jnp.float32) - m0) * scale
    flag = jnp.where(k == TB, 1.0, 0.0)
    out_ref[...] = sm + cp_ref[...] * flag


def kernel(hidden, attn, src_map, W, b, w_copy, b_copy, pad_idx):
    b_m = b.at[pad_idx].set(NEG)
    b_ext = jnp.concatenate(
        [b_m, jnp.zeros((VP - V,), jnp.float32)]).reshape(1, VP)
    wc = w_copy.reshape(D, 1)
    bc = b_copy.reshape(1, 1)

    pc, cp = pl.pallas_call(
        _gate_copy_kernel,
        grid=(B,),
        in_specs=[
            pl.BlockSpec((T, D), lambda i: (i, 0)),
            pl.BlockSpec((T, S), lambda i: (i, 0)),
            pl.BlockSpec((1, S, C), lambda i: (i, 0, 0)),
            pl.BlockSpec((D, 1), lambda i: (0, 0)),
            pl.BlockSpec((1, 1), lambda i: (0, 0)),
        ],
        out_specs=[
            pl.BlockSpec((T, 128), lambda i: (i, 0)),
            pl.BlockSpec((T, BNC), lambda i: (i, 0)),
        ],
        out_shape=[
            jax.ShapeDtypeStruct((BT, 128), jnp.float32),
            jax.ShapeDtypeStruct((BT, BNC), jnp.float32),
        ],
        compiler_params=pltpu.CompilerParams(
            dimension_semantics=("parallel",)),
    )(hidden, attn, src_map, wc, bc)

    lg, m, s = pl.pallas_call(
        _logits_kernel,
        grid=(BT // BM, KB),
        in_specs=[
            pl.BlockSpec((BM, D), lambda i, k: (i, 0)),
            pl.BlockSpec((BN, D), lambda i, k: (k, 0)),
            pl.BlockSpec((1, BN), lambda i, k: (0, k)),
        ],
        out_specs=[
            pl.BlockSpec((BM, BN), lambda i, k: (i, k)),
            pl.BlockSpec((BM, 128), lambda i, k: (i, 0)),
            pl.BlockSpec((BM, 128), lambda i, k: (i, 0)),
        ],
        out_shape=[
            jax.ShapeDtypeStruct((BT, VP), jnp.bfloat16),
            jax.ShapeDtypeStruct((BT, 128), jnp.float32),
            jax.ShapeDtypeStruct((BT, 128), jnp.float32),
        ],
        scratch_shapes=[
            pltpu.VMEM((BM, 1), jnp.float32),
            pltpu.VMEM((BM, 1), jnp.float32),
        ],
        compiler_params=pltpu.CompilerParams(
            dimension_semantics=("parallel", "arbitrary"),
            vmem_limit_bytes=52 * 1024 * 1024),
    )(hidden, W, b_ext)

    out = pl.pallas_call(
        _finalize_kernel,
        grid=(BT // BM, KC),
        in_specs=[
            pl.BlockSpec((BM, BNC), lambda i, k: (i, k)),
            pl.BlockSpec((BM, 128), lambda i, k: (i, 0)),
            pl.BlockSpec((BM, 128), lambda i, k: (i, 0)),
            pl.BlockSpec((BM, 128), lambda i, k: (i, 0)),
            pl.BlockSpec((BM, BNC), lambda i, k: (i, 0)),
        ],
        out_specs=pl.BlockSpec((BM, BNC), lambda i, k: (i, k)),
        out_shape=jax.ShapeDtypeStruct((BT, V + C), jnp.float32),
        compiler_params=pltpu.CompilerParams(
            dimension_semantics=("parallel", "arbitrary"),
            vmem_limit_bytes=52 * 1024 * 1024),
    )(lg, m, s, pc, cp)
    return out


# speedup vs baseline: 1.1126x; 1.0007x over previous
import jax
import jax.numpy as jnp
from jax.experimental import pallas as pl
from jax.experimental.pallas import tpu as pltpu

B, T, S, C, V, D = 16, 128, 512, 512, 50000, 1024
BT = B * T
NEG = -1e30

BM = 1024
BN = 1280
VP = 51200
KB = VP // BN

BNC = 2048
KC = (V + C + BNC - 1) // BNC
TB = V // BNC
OFF = V - TB * BNC


def _gate_copy_kernel(hid_ref, attn_ref, sm_ref, wc_ref, bc_ref,
                      pc_ref, cp_ref):
    pc = jax.nn.sigmoid(
        jnp.dot(hid_ref[...], wc_ref[...], preferred_element_type=jnp.float32)
        + bc_ref[0, 0])
    pc_ref[...] = jnp.broadcast_to(pc, (T, 128))
    mula = attn_ref[...] * pc
    cp = jnp.dot(mula, sm_ref[0], preferred_element_type=jnp.float32)
    cp_ref[...] = jnp.concatenate(
        [jnp.zeros((T, OFF), jnp.float32), cp,
         jnp.zeros((T, BNC - OFF - C), jnp.float32)], axis=1)


def _logits_kernel(hid_ref, w_ref, b_ref, lg_ref, m_ref, s_ref, m_s, s_s):
    k = pl.program_id(0)

    @pl.when(k == 0)
    def _():
        m_s[...] = jnp.full((BT, 1), NEG, jnp.float32)
        s_s[...] = jnp.zeros((BT, 1), jnp.float32)

    l = jax.lax.dot_general(hid_ref[...], w_ref[...],
                            (((1,), (1,)), ((), ())),
                            preferred_element_type=jnp.float32)
    l = l + b_ref[...]

    def _update(lv):
        tmax = jnp.max(lv, axis=1, keepdims=True)
        m_old = m_s[...]
        m_new = jnp.maximum(m_old, tmax)
        s_new = (s_s[...] * jnp.exp(m_old - m_new)
                 + jnp.sum(jnp.exp(lv - m_new), axis=1, keepdims=True))
        m_s[...] = m_new
        s_s[...] = s_new
        lg_ref[...] = lv.astype(jnp.bfloat16)
        return m_new, s_new

    @pl.when(k < KB - 1)
    def _():
        _update(l)

    @pl.when(k == KB - 1)
    def _():
        col = jax.lax.broadcasted_iota(jnp.int32, (1, BN), 1) + k * BN
        m_new, s_new = _update(jnp.where(col >= V, NEG, l))
        m_ref[...] = jnp.broadcast_to(m_new, (BT, 128))
        s_ref[...] = jnp.broadcast_to(s_new, (BT, 128))


def _finalize_kernel(lg_ref, m_ref, s_ref, pc_ref, cp_ref, out_ref):
    k = pl.program_id(1)
    m0 = jnp.max(m_ref[...], axis=1, keepdims=True)
    s0 = jnp.max(s_ref[...], axis=1, keepdims=True)
    pc0 = jnp.max(pc_ref[...], axis=1, keepdims=True)
    scale = (1.0 - pc0) / s0
    sm = jnp.exp(lg_ref[...].astype(jnp.float32) - m0) * scale
    flag = jnp.where(k == TB, 1.0, 0.0)
    out_ref[...] = sm + cp_ref[...] * flag


def kernel(hidden, attn, src_map, W, b, w_copy, b_copy, pad_idx):
    b_m = b.at[pad_idx].set(NEG)
    b_ext = jnp.concatenate(
        [b_m, jnp.zeros((VP - V,), jnp.float32)]).reshape(1, VP)
    wc = w_copy.reshape(D, 1)
    bc = b_copy.reshape(1, 1)

    pc, cp = pl.pallas_call(
        _gate_copy_kernel,
        grid=(B,),
        in_specs=[
            pl.BlockSpec((T, D), lambda i: (i, 0)),
            pl.BlockSpec((T, S), lambda i: (i, 0)),
            pl.BlockSpec((1, S, C), lambda i: (i, 0, 0)),
            pl.BlockSpec((D, 1), lambda i: (0, 0)),
            pl.BlockSpec((1, 1), lambda i: (0, 0)),
        ],
        out_specs=[
            pl.BlockSpec((T, 128), lambda i: (i, 0)),
            pl.BlockSpec((T, BNC), lambda i: (i, 0)),
        ],
        out_shape=[
            jax.ShapeDtypeStruct((BT, 128), jnp.float32),
            jax.ShapeDtypeStruct((BT, BNC), jnp.float32),
        ],
        compiler_params=pltpu.CompilerParams(
            dimension_semantics=("parallel",)),
    )(hidden, attn, src_map, wc, bc)

    lg, m, s = pl.pallas_call(
        _logits_kernel,
        grid=(KB,),
        in_specs=[
            pl.BlockSpec((BT, D), lambda k: (0, 0)),
            pl.BlockSpec((BN, D), lambda k: (k, 0)),
            pl.BlockSpec((1, BN), lambda k: (0, k)),
        ],
        out_specs=[
            pl.BlockSpec((BT, BN), lambda k: (0, k)),
            pl.BlockSpec((BT, 128), lambda k: (0, 0)),
            pl.BlockSpec((BT, 128), lambda k: (0, 0)),
        ],
        out_shape=[
            jax.ShapeDtypeStruct((BT, VP), jnp.bfloat16),
            jax.ShapeDtypeStruct((BT, 128), jnp.float32),
            jax.ShapeDtypeStruct((BT, 128), jnp.float32),
        ],
        scratch_shapes=[
            pltpu.VMEM((BT, 1), jnp.float32),
            pltpu.VMEM((BT, 1), jnp.float32),
        ],
        compiler_params=pltpu.CompilerParams(
            dimension_semantics=("arbitrary",),
            vmem_limit_bytes=56 * 1024 * 1024),
    )(hidden, W, b_ext)

    out = pl.pallas_call(
        _finalize_kernel,
        grid=(BT // BM, KC),
        in_specs=[
            pl.BlockSpec((BM, BNC), lambda i, k: (i, k)),
            pl.BlockSpec((BM, 128), lambda i, k: (i, 0)),
            pl.BlockSpec((BM, 128), lambda i, k: (i, 0)),
            pl.BlockSpec((BM, 128), lambda i, k: (i, 0)),
            pl.BlockSpec((BM, BNC), lambda i, k: (i, 0)),
        ],
        out_specs=pl.BlockSpec((BM, BNC), lambda i, k: (i, k)),
        out_shape=jax.ShapeDtypeStruct((BT, V + C), jnp.float32),
        compiler_params=pltpu.CompilerParams(
            dimension_semantics=("parallel", "arbitrary"),
            vmem_limit_bytes=52 * 1024 * 1024),
    )(lg, m, s, pc, cp)
    return out
